# baseline (speedup 1.0000x reference)
import jax
import jax.numpy as jnp
from jax.experimental import pallas as pl
from jax.experimental.pallas import tpu as pltpu

_B, _C, _H, _W = 2, 2, 224, 224
_INF = 1e12
_FALLBACK = (_H + _W) / 4.0


def _neighbor_sum(img):
    zr = jnp.zeros((1, _W), jnp.float32)
    zc = jnp.zeros((_H, 1), jnp.float32)
    up = jnp.concatenate([img[1:, :], zr], axis=0)
    down = jnp.concatenate([zr, img[:-1, :]], axis=0)
    left = jnp.concatenate([img[:, 1:], zc], axis=1)
    right = jnp.concatenate([zc, img[:, :-1]], axis=1)
    return img + up + down + left + right


def _hd_kernel(pred_ref, lab_ref, out_ref, pad_ref, run_ref):
    p0 = pred_ref[0, 0]
    p1 = pred_ref[0, 1]
    img_b = lab_ref[0, 0].astype(jnp.float32)
    img_a = (p1 > p0).astype(jnp.float32)

    boundary_a = img_a * (5.0 - _neighbor_sum(img_a)) * (1.0 - img_b)
    boundary_b = img_b * (5.0 - _neighbor_sum(img_b))
    ma = (boundary_a > 0.0).astype(jnp.float32)
    mb = (boundary_b > 0.0).astype(jnp.float32)

    inf_128 = jnp.full((128, _W), jnp.float32(_INF), jnp.float32)
    pad_ref[_H - 128:_H, :] = inf_128
    pad_ref[2 * _H:2 * _H + 128, :] = inf_128

    d = jnp.where(mb > 0.0, jnp.float32(0.0), jnp.float32(_INF))
    for k in (1, 2, 4, 8):
        pad_ref[_H:2 * _H, :] = d
        up = pad_ref[_H - k:2 * _H - k, :]
        dn = pad_ref[_H + k:2 * _H + k, :]
        d = jnp.minimum(d, jnp.minimum(up, dn) + jnp.float32(k))
    run_ref[...] = d
    md = jnp.max(d)

    @pl.when(md > 15.0)
    def _():
        dd = run_ref[...]
        for k in (16, 32, 64, 128):
            pad_ref[_H:2 * _H, :] = dd
            up = pad_ref[_H - k:2 * _H - k, :]
            dn = pad_ref[_H + k:2 * _H + k, :]
            dd = jnp.minimum(dd, jnp.minimum(up, dn) + jnp.float32(k))
        run_ref[...] = dd

    d = run_ref[...]
    d1 = d * d

    d1t = d1.T
    mat = ma.T
    pad_ref[_H:2 * _H, :] = d1t

    w0 = pad_ref[_H - 4:_H + 228, :]
    r = d1t
    for s in (1, 2, 3, 4):
        cand = jnp.minimum(w0[4 - s:228 - s, :], w0[4 + s:228 + s, :])
        r = jnp.minimum(r, cand + jnp.float32(s * s))
    run_ref[...] = r
    mm1 = jnp.max(jnp.where(mat > 0.0, r, -jnp.float32(_INF)))

    @pl.when(mm1 > 25.0)
    def _():
        inf_96 = jnp.full((96, _W), jnp.float32(_INF), jnp.float32)
        pad_ref[0:96, :] = inf_96
        pad_ref[2 * _H + 128:3 * _H, :] = inf_96

    def cond(carry):
        t, mm = carry
        s0 = jnp.maximum(8 * t + 1, 5)
        return (8 * t + 1 < _W) & (mm > (s0 * s0).astype(jnp.float32))

    def body(carry):
        t, _ = carry
        wp = pad_ref[pl.ds(8 * (_H // 8 - 1 - t), _H + 8), :]
        wm = pad_ref[pl.ds(_H + 8 * t, _H + 8), :]
        r = run_ref[...]
        for k in range(8):
            sf = (8 * t + (1 + k)).astype(jnp.float32)
            cand = jnp.minimum(wp[7 - k:7 - k + _H, :],
                               wm[1 + k:1 + k + _H, :])
            r = jnp.minimum(r, cand + sf * sf)
        run_ref[...] = r
        mm = jnp.max(jnp.where(mat > 0.0, r, -jnp.float32(_INF)))
        return t + 1, mm

    _, mm_final = jax.lax.while_loop(cond, body, (jnp.int32(0), mm1))

    hd = jnp.sqrt(jnp.maximum(mm_final, 0.0))
    has_both = (mm_final >= 0.0) & (mm_final < 1e11)
    val = jnp.where(has_both, hd, jnp.float32(_FALLBACK))

    lane = jax.lax.broadcasted_iota(jnp.int32, (1, _C + 2), 1)
    row = jnp.where(lane == 1, val,
                    jnp.where(lane == 2, val * 0.5, jnp.float32(0.0)))
    b = pl.program_id(0)

    @pl.when(b == 0)
    def _():
        out_ref[0:1, :] = row

    @pl.when(b == 1)
    def _():
        out_ref[1:2, :] = row


def kernel(predictions, labels):
    res = pl.pallas_call(
        _hd_kernel,
        grid=(_B,),
        in_specs=[
            pl.BlockSpec((1, _C, _H, _W), lambda b: (b, 0, 0, 0)),
            pl.BlockSpec((1, 1, _H, _W), lambda b: (b, 1, 0, 0)),
        ],
        out_specs=pl.BlockSpec((_B, _C + 2), lambda b: (0, 0)),
        out_shape=jax.ShapeDtypeStruct((_B, _C + 2), jnp.float32),
        scratch_shapes=[
            pltpu.VMEM((3 * _H, _W), jnp.float32),
            pltpu.VMEM((_H, _W), jnp.float32),
        ],
    )(predictions, labels)
    return res


# speedup vs baseline: 1.0540x; 1.0540x over previous
import jax
import jax.numpy as jnp
from jax.experimental import pallas as pl
from jax.experimental.pallas import tpu as pltpu

_B, _C, _H, _W = 2, 2, 224, 224
_INF = 1e12
_FALLBACK = (_H + _W) / 4.0


def _neighbor_sum(img):
    zr = jnp.zeros((1, _W), jnp.float32)
    zc = jnp.zeros((_H, 1), jnp.float32)
    up = jnp.concatenate([img[1:, :], zr], axis=0)
    down = jnp.concatenate([zr, img[:-1, :]], axis=0)
    left = jnp.concatenate([img[:, 1:], zc], axis=1)
    right = jnp.concatenate([zc, img[:, :-1]], axis=1)
    return img + up + down + left + right


def _hd_kernel(pred_ref, lab_ref, out_ref, pad_ref, run_ref):
    p0 = pred_ref[0, 0]
    p1 = pred_ref[0, 1]
    img_b = lab_ref[0, 0].astype(jnp.float32)
    img_a = (p1 > p0).astype(jnp.float32)

    boundary_a = img_a * (5.0 - _neighbor_sum(img_a)) * (1.0 - img_b)
    boundary_b = img_b * (5.0 - _neighbor_sum(img_b))
    ma = (boundary_a > 0.0).astype(jnp.float32)
    mb = (boundary_b > 0.0).astype(jnp.float32)

    inf_128 = jnp.full((128, _W), jnp.float32(_INF), jnp.float32)
    pad_ref[_H - 128:_H, :] = inf_128
    pad_ref[2 * _H:2 * _H + 128, :] = inf_128

    d = jnp.where(mb > 0.0, jnp.float32(0.0), jnp.float32(_INF))
    for k in (1, 2, 4, 8, 16, 32, 64, 128):
        pad_ref[_H:2 * _H, :] = d
        up = pad_ref[_H - k:2 * _H - k, :]
        dn = pad_ref[_H + k:2 * _H + k, :]
        d = jnp.minimum(d, jnp.minimum(up, dn) + jnp.float32(k))
    d1 = d * d

    d1t = d1.T
    mat = ma.T
    pad_ref[_H:2 * _H, :] = d1t

    w0 = pad_ref[_H - 4:_H + 228, :]
    r = d1t
    for s in (1, 2, 3, 4):
        cand = jnp.minimum(w0[4 - s:228 - s, :], w0[4 + s:228 + s, :])
        r = jnp.minimum(r, cand + jnp.float32(s * s))
    run_ref[...] = r
    mm1 = jnp.max(jnp.where(mat > 0.0, r, -jnp.float32(_INF)))

    @pl.when(mm1 > 25.0)
    def _():
        inf_96 = jnp.full((96, _W), jnp.float32(_INF), jnp.float32)
        pad_ref[0:96, :] = inf_96
        pad_ref[2 * _H + 128:3 * _H, :] = inf_96

    def cond(carry):
        t, mm = carry
        s0 = jnp.maximum(8 * t + 1, 5)
        return (8 * t + 1 < _W) & (mm > (s0 * s0).astype(jnp.float32))

    def body(carry):
        t, _ = carry
        wp = pad_ref[pl.ds(8 * (_H // 8 - 1 - t), _H + 8), :]
        wm = pad_ref[pl.ds(_H + 8 * t, _H + 8), :]
        r = run_ref[...]
        for k in range(8):
            sf = (8 * t + (1 + k)).astype(jnp.float32)
            cand = jnp.minimum(wp[7 - k:7 - k + _H, :],
                               wm[1 + k:1 + k + _H, :])
            r = jnp.minimum(r, cand + sf * sf)
        run_ref[...] = r
        mm = jnp.max(jnp.where(mat > 0.0, r, -jnp.float32(_INF)))
        return t + 1, mm

    _, mm_final = jax.lax.while_loop(cond, body, (jnp.int32(0), mm1))

    hd = jnp.sqrt(jnp.maximum(mm_final, 0.0))
    has_both = (mm_final >= 0.0) & (mm_final < 1e11)
    val = jnp.where(has_both, hd, jnp.float32(_FALLBACK))

    lane = jax.lax.broadcasted_iota(jnp.int32, (1, _C + 2), 1)
    row = jnp.where(lane == 1, val,
                    jnp.where(lane == 2, val * 0.5, jnp.float32(0.0)))
    b = pl.program_id(0)

    @pl.when(b == 0)
    def _():
        out_ref[0:1, :] = row

    @pl.when(b == 1)
    def _():
        out_ref[1:2, :] = row


def kernel(predictions, labels):
    res = pl.pallas_call(
        _hd_kernel,
        grid=(_B,),
        in_specs=[
            pl.BlockSpec((1, _C, _H, _W), lambda b: (b, 0, 0, 0)),
            pl.BlockSpec((1, 1, _H, _W), lambda b: (b, 1, 0, 0)),
        ],
        out_specs=pl.BlockSpec((_B, _C + 2), lambda b: (0, 0)),
        out_shape=jax.ShapeDtypeStruct((_B, _C + 2), jnp.float32),
        scratch_shapes=[
            pltpu.VMEM((3 * _H, _W), jnp.float32),
            pltpu.VMEM((_H, _W), jnp.float32),
        ],
    )(predictions, labels)
    return res
